# baseline (speedup 1.0000x reference)
import functools

import jax
import jax.numpy as jnp
import numpy as np
from jax.experimental import pallas as pl

_K = 20
_EPS = 1e-3
_TILE = 64


_NT = (((1,), (1,)), ((), ()))


def _edgeconv_kernel(x_full_ref, x_tile_ref, W1_ref, b1_ref, s1_ref,
                     t1_ref, W2_ref, b2_ref, s2_ref, t2_ref, out_ref, *, N, C, F):
    f32 = jnp.float32
    xb = x_full_ref[0]
    x_R = x_tile_ref[0]

    ones_row = jnp.ones((1, C), dtype=f32)
    sq = xb * xb
    sq_hi = sq.astype(jnp.bfloat16)
    sq_lo = (sq - sq_hi.astype(f32)).astype(jnp.bfloat16)
    ones_bf = jnp.ones((1, C), dtype=jnp.bfloat16)
    xx_col = (jax.lax.dot_general(ones_bf, sq_hi, _NT,
                                  preferred_element_type=f32) +
              jax.lax.dot_general(ones_bf, sq_lo, _NT,
                                  preferred_element_type=f32))
    nz_col = jax.lax.dot_general(ones_row, (xb != 0.0).astype(f32), _NT,
                                 preferred_element_type=f32)
    mask_col = (nz_col > 0.0).astype(f32)
    xx_row = jnp.sum(x_R * x_R, axis=1, keepdims=True)
    mask_row = (jnp.max(jnp.abs(x_R), axis=1, keepdims=True) > 0.0)
    mask_row = mask_row.astype(f32)

    G = jax.lax.dot_general(x_R, xb, _NT,
                            preferred_element_type=f32)
    D = (xx_row - 2.0 * G) + xx_col
    D = D + (1.0 - mask_row * mask_col) * 1000000.0

    W1a = W1_ref[0:C, :]
    W1b = W1_ref[C:2 * C, :]
    q = jnp.dot(xb, W1b, preferred_element_type=f32)
    q_hi = q.astype(jnp.bfloat16)
    q_lo = (q - q_hi.astype(f32)).astype(jnp.bfloat16)
    p_R = jnp.dot(x_R, W1a - W1b, preferred_element_type=f32)
    pre_base = p_R + b1_ref[0]
    s1 = s1_ref[0]
    t1 = t1_ref[0]
    W2 = W2_ref[...]
    b2 = b2_ref[0]
    s2 = s2_ref[0]
    t2 = t2_ref[0]

    iota_col = jax.lax.broadcasted_iota(jnp.int32, (_TILE, N), 1)

    acc = jnp.full((_TILE, F), -jnp.inf, dtype=f32)
    m = jnp.min(D, axis=1, keepdims=True)
    for k in range(_K):
        idx = jnp.min(jnp.where(D == m, iota_col, N), axis=1,
                      keepdims=True)
        oh = iota_col == idx
        if k < _K - 1:
            D = jnp.where(oh, jnp.inf, D)
            m = jnp.min(D, axis=1, keepdims=True)
        oh_bf = oh.astype(jnp.bfloat16)
        nbr = (jax.lax.dot_general(oh_bf, q_hi, (((1,), (0,)), ((), ())),
                                   preferred_element_type=f32) +
               jax.lax.dot_general(oh_bf, q_lo, (((1,), (0,)), ((), ())),
                                   preferred_element_type=f32))
        pre1 = pre_base + nbr * mask_row
        y1 = jnp.maximum(pre1, 0.0) * s1 + t1
        pre2 = jnp.dot(y1, W2, preferred_element_type=f32) + b2
        y2 = jnp.maximum(pre2, 0.0) * s2 + t2
        acc = jnp.maximum(acc, y2)
    out_ref[0] = acc


def _edgeconv_shard(x, W1, b1r, s1, t1, W2, b2r, s2, t2):
    B, N, C = x.shape
    F = W1.shape[1]
    grid = (B, N // _TILE)
    out = pl.pallas_call(
        functools.partial(_edgeconv_kernel, N=N, C=C, F=F),
        grid=grid,
        in_specs=[
            pl.BlockSpec((1, N, C), lambda b, t: (b, 0, 0)),
            pl.BlockSpec((1, _TILE, C), lambda b, t: (b, t, 0)),
            pl.BlockSpec((2 * C, F), lambda b, t: (0, 0)),
            pl.BlockSpec((1, F), lambda b, t: (0, 0)),
            pl.BlockSpec((1, F), lambda b, t: (0, 0)),
            pl.BlockSpec((1, F), lambda b, t: (0, 0)),
            pl.BlockSpec((F, F), lambda b, t: (0, 0)),
            pl.BlockSpec((1, F), lambda b, t: (0, 0)),
            pl.BlockSpec((1, F), lambda b, t: (0, 0)),
            pl.BlockSpec((1, F), lambda b, t: (0, 0)),
        ],
        out_specs=pl.BlockSpec((1, _TILE, F), lambda b, t: (b, t, 0)),
        out_shape=jax.ShapeDtypeStruct((B, N, F), jnp.float32),
    )(x, x, W1, b1r, s1, t1, W2, b2r, s2, t2)
    return out


def kernel(x, W1, b1, gamma1, beta1, mean1, var1, W2, b2, gamma2, beta2,
           mean2, var2):
    B, N, C = x.shape
    F = W1.shape[1]
    s1 = (gamma1 / jnp.sqrt(var1 + _EPS)).reshape(1, F)
    t1 = (beta1 - mean1 * (gamma1 / jnp.sqrt(var1 + _EPS))).reshape(1, F)
    s2 = (gamma2 / jnp.sqrt(var2 + _EPS)).reshape(1, F)
    t2 = (beta2 - mean2 * (gamma2 / jnp.sqrt(var2 + _EPS))).reshape(1, F)
    b1r = b1.reshape(1, F)
    b2r = b2.reshape(1, F)
    args = (x, W1, b1r, s1, t1, W2, b2r, s2, t2)

    devs = jax.devices()
    nd = max(d for d in (1, 2, 4) if d <= len(devs) and B % d == 0)
    if nd == 1:
        return _edgeconv_shard(*args)
    mesh = jax.sharding.Mesh(np.array(devs[:nd]), ("d",))
    P = jax.sharding.PartitionSpec
    rep = P()
    fn = jax.shard_map(
        _edgeconv_shard,
        mesh=mesh,
        in_specs=(P("d"), rep, rep, rep, rep, rep, rep, rep, rep),
        out_specs=P("d"),
        check_vma=False,
    )
    return fn(*args)


# speedup vs baseline: 1.0615x; 1.0615x over previous
import functools

import jax
import jax.numpy as jnp
import numpy as np
from jax.experimental import pallas as pl

_K = 20
_EPS = 1e-3
_TILE = 128


_NT = (((1,), (1,)), ((), ()))


def _edgeconv_kernel(x_full_ref, x_tile_ref, W1_ref, b1_ref, s1_ref,
                     t1_ref, W2_ref, b2_ref, s2_ref, t2_ref, out_ref, *, N, C, F):
    f32 = jnp.float32
    xb = x_full_ref[0]
    x_R = x_tile_ref[0]

    ones_row = jnp.ones((1, C), dtype=f32)
    sq = xb * xb
    sq_hi = sq.astype(jnp.bfloat16)
    sq_lo = (sq - sq_hi.astype(f32)).astype(jnp.bfloat16)
    ones_bf = jnp.ones((1, C), dtype=jnp.bfloat16)
    xx_col = (jax.lax.dot_general(ones_bf, sq_hi, _NT,
                                  preferred_element_type=f32) +
              jax.lax.dot_general(ones_bf, sq_lo, _NT,
                                  preferred_element_type=f32))
    nz_col = jax.lax.dot_general(ones_row, (xb != 0.0).astype(f32), _NT,
                                 preferred_element_type=f32)
    mask_col = (nz_col > 0.0).astype(f32)
    xx_row = jnp.sum(x_R * x_R, axis=1, keepdims=True)
    mask_row = (jnp.max(jnp.abs(x_R), axis=1, keepdims=True) > 0.0)
    mask_row = mask_row.astype(f32)

    G = jax.lax.dot_general(x_R, xb, _NT,
                            preferred_element_type=f32)
    D = (xx_row - 2.0 * G) + xx_col
    D = D + (1.0 - mask_row * mask_col) * 1000000.0

    W1a = W1_ref[0:C, :]
    W1b = W1_ref[C:2 * C, :]
    q = jnp.dot(xb, W1b, preferred_element_type=f32)
    q_hi = q.astype(jnp.bfloat16)
    q_lo = (q - q_hi.astype(f32)).astype(jnp.bfloat16)
    p_R = jnp.dot(x_R, W1a - W1b, preferred_element_type=f32)
    pre_base = p_R + b1_ref[0]
    s1 = s1_ref[0]
    t1 = t1_ref[0]
    W2 = W2_ref[...]
    b2 = b2_ref[0]
    s2 = s2_ref[0]
    t2 = t2_ref[0]

    iota_col = jax.lax.broadcasted_iota(jnp.int32, (_TILE, N), 1)

    acc = jnp.full((_TILE, F), -jnp.inf, dtype=f32)
    m = jnp.min(D, axis=1, keepdims=True)
    for k in range(_K):
        idx = jnp.min(jnp.where(D == m, iota_col, N), axis=1,
                      keepdims=True)
        oh = iota_col == idx
        if k < _K - 1:
            D = jnp.where(oh, jnp.inf, D)
            m = jnp.min(D, axis=1, keepdims=True)
        oh_bf = oh.astype(jnp.bfloat16)
        nbr = (jax.lax.dot_general(oh_bf, q_hi, (((1,), (0,)), ((), ())),
                                   preferred_element_type=f32) +
               jax.lax.dot_general(oh_bf, q_lo, (((1,), (0,)), ((), ())),
                                   preferred_element_type=f32))
        pre1 = pre_base + nbr * mask_row
        y1 = jnp.maximum(pre1, 0.0) * s1 + t1
        pre2 = jnp.dot(y1, W2, preferred_element_type=f32) + b2
        y2 = jnp.maximum(pre2, 0.0) * s2 + t2
        acc = jnp.maximum(acc, y2)
    out_ref[0] = acc


def _edgeconv_shard(x, W1, b1r, s1, t1, W2, b2r, s2, t2):
    B, N, C = x.shape
    F = W1.shape[1]
    grid = (B, N // _TILE)
    out = pl.pallas_call(
        functools.partial(_edgeconv_kernel, N=N, C=C, F=F),
        grid=grid,
        in_specs=[
            pl.BlockSpec((1, N, C), lambda b, t: (b, 0, 0)),
            pl.BlockSpec((1, _TILE, C), lambda b, t: (b, t, 0)),
            pl.BlockSpec((2 * C, F), lambda b, t: (0, 0)),
            pl.BlockSpec((1, F), lambda b, t: (0, 0)),
            pl.BlockSpec((1, F), lambda b, t: (0, 0)),
            pl.BlockSpec((1, F), lambda b, t: (0, 0)),
            pl.BlockSpec((F, F), lambda b, t: (0, 0)),
            pl.BlockSpec((1, F), lambda b, t: (0, 0)),
            pl.BlockSpec((1, F), lambda b, t: (0, 0)),
            pl.BlockSpec((1, F), lambda b, t: (0, 0)),
        ],
        out_specs=pl.BlockSpec((1, _TILE, F), lambda b, t: (b, t, 0)),
        out_shape=jax.ShapeDtypeStruct((B, N, F), jnp.float32),
    )(x, x, W1, b1r, s1, t1, W2, b2r, s2, t2)
    return out


def kernel(x, W1, b1, gamma1, beta1, mean1, var1, W2, b2, gamma2, beta2,
           mean2, var2):
    B, N, C = x.shape
    F = W1.shape[1]
    s1 = (gamma1 / jnp.sqrt(var1 + _EPS)).reshape(1, F)
    t1 = (beta1 - mean1 * (gamma1 / jnp.sqrt(var1 + _EPS))).reshape(1, F)
    s2 = (gamma2 / jnp.sqrt(var2 + _EPS)).reshape(1, F)
    t2 = (beta2 - mean2 * (gamma2 / jnp.sqrt(var2 + _EPS))).reshape(1, F)
    b1r = b1.reshape(1, F)
    b2r = b2.reshape(1, F)
    args = (x, W1, b1r, s1, t1, W2, b2r, s2, t2)

    devs = jax.devices()
    nd = max(d for d in (1, 2, 4) if d <= len(devs) and B % d == 0)
    if nd == 1:
        return _edgeconv_shard(*args)
    mesh = jax.sharding.Mesh(np.array(devs[:nd]), ("d",))
    P = jax.sharding.PartitionSpec
    rep = P()
    fn = jax.shard_map(
        _edgeconv_shard,
        mesh=mesh,
        in_specs=(P("d"), rep, rep, rep, rep, rep, rep, rep, rep),
        out_specs=P("d"),
        check_vma=False,
    )
    return fn(*args)


# speedup vs baseline: 1.0633x; 1.0018x over previous
import functools

import jax
import jax.numpy as jnp
import numpy as np
from jax.experimental import pallas as pl
from jax.experimental.pallas import tpu as pltpu

_K = 20
_EPS = 1e-3
_TILE = 128


_NT = (((1,), (1,)), ((), ()))


def _edgeconv_kernel(x_full_ref, x_tile_ref, W1_ref, b1_ref, s1_ref,
                     t1_ref, W2_ref, b2_ref, s2_ref, t2_ref, out_ref, *, N, C, F):
    f32 = jnp.float32
    xb = x_full_ref[0]
    x_R = x_tile_ref[0]

    ones_row = jnp.ones((1, C), dtype=f32)
    sq = xb * xb
    sq_hi = sq.astype(jnp.bfloat16)
    sq_lo = (sq - sq_hi.astype(f32)).astype(jnp.bfloat16)
    ones_bf = jnp.ones((1, C), dtype=jnp.bfloat16)
    xx_col = (jax.lax.dot_general(ones_bf, sq_hi, _NT,
                                  preferred_element_type=f32) +
              jax.lax.dot_general(ones_bf, sq_lo, _NT,
                                  preferred_element_type=f32))
    nz_col = jax.lax.dot_general(ones_row, (xb != 0.0).astype(f32), _NT,
                                 preferred_element_type=f32)
    mask_col = (nz_col > 0.0).astype(f32)
    xx_row = jnp.sum(x_R * x_R, axis=1, keepdims=True)
    mask_row = (jnp.max(jnp.abs(x_R), axis=1, keepdims=True) > 0.0)
    mask_row = mask_row.astype(f32)

    G = jax.lax.dot_general(x_R, xb, _NT,
                            preferred_element_type=f32)
    D = (xx_row - 2.0 * G) + xx_col
    D = D + (1.0 - mask_row * mask_col) * 1000000.0

    W1a = W1_ref[0:C, :]
    W1b = W1_ref[C:2 * C, :]
    q = jnp.dot(xb, W1b, preferred_element_type=f32)
    q_hi = q.astype(jnp.bfloat16)
    q_lo = (q - q_hi.astype(f32)).astype(jnp.bfloat16)
    p_R = jnp.dot(x_R, W1a - W1b, preferred_element_type=f32)
    pre_base = p_R + b1_ref[0]
    s1 = s1_ref[0]
    t1 = t1_ref[0]
    W2 = W2_ref[...]
    b2 = b2_ref[0]
    s2 = s2_ref[0]
    t2 = t2_ref[0]

    iota_col = jax.lax.broadcasted_iota(jnp.int32, (_TILE, N), 1)

    acc = jnp.full((_TILE, F), -jnp.inf, dtype=f32)
    m = jnp.min(D, axis=1, keepdims=True)
    for k in range(_K):
        idx = jnp.min(jnp.where(D == m, iota_col, N), axis=1,
                      keepdims=True)
        oh = iota_col == idx
        if k < _K - 1:
            D = jnp.where(oh, jnp.inf, D)
            m = jnp.min(D, axis=1, keepdims=True)
        oh_bf = oh.astype(jnp.bfloat16)
        nbr = (jax.lax.dot_general(oh_bf, q_hi, (((1,), (0,)), ((), ())),
                                   preferred_element_type=f32) +
               jax.lax.dot_general(oh_bf, q_lo, (((1,), (0,)), ((), ())),
                                   preferred_element_type=f32))
        pre1 = pre_base + nbr * mask_row
        y1 = jnp.maximum(pre1, 0.0) * s1 + t1
        pre2 = jnp.dot(y1, W2, preferred_element_type=f32) + b2
        y2 = jnp.maximum(pre2, 0.0) * s2 + t2
        acc = jnp.maximum(acc, y2)
    out_ref[0] = acc


def _edgeconv_shard(x, W1, b1r, s1, t1, W2, b2r, s2, t2):
    B, N, C = x.shape
    F = W1.shape[1]
    grid = (B, N // _TILE)
    out = pl.pallas_call(
        functools.partial(_edgeconv_kernel, N=N, C=C, F=F),
        grid=grid,
        in_specs=[
            pl.BlockSpec((1, N, C), lambda b, t: (b, 0, 0)),
            pl.BlockSpec((1, _TILE, C), lambda b, t: (b, t, 0)),
            pl.BlockSpec((2 * C, F), lambda b, t: (0, 0)),
            pl.BlockSpec((1, F), lambda b, t: (0, 0)),
            pl.BlockSpec((1, F), lambda b, t: (0, 0)),
            pl.BlockSpec((1, F), lambda b, t: (0, 0)),
            pl.BlockSpec((F, F), lambda b, t: (0, 0)),
            pl.BlockSpec((1, F), lambda b, t: (0, 0)),
            pl.BlockSpec((1, F), lambda b, t: (0, 0)),
            pl.BlockSpec((1, F), lambda b, t: (0, 0)),
        ],
        out_specs=pl.BlockSpec((1, _TILE, F), lambda b, t: (b, t, 0)),
        out_shape=jax.ShapeDtypeStruct((B, N, F), jnp.float32),
        compiler_params=pltpu.CompilerParams(
            dimension_semantics=("arbitrary", "parallel")),
    )(x, x, W1, b1r, s1, t1, W2, b2r, s2, t2)
    return out


def kernel(x, W1, b1, gamma1, beta1, mean1, var1, W2, b2, gamma2, beta2,
           mean2, var2):
    B, N, C = x.shape
    F = W1.shape[1]
    s1 = (gamma1 / jnp.sqrt(var1 + _EPS)).reshape(1, F)
    t1 = (beta1 - mean1 * (gamma1 / jnp.sqrt(var1 + _EPS))).reshape(1, F)
    s2 = (gamma2 / jnp.sqrt(var2 + _EPS)).reshape(1, F)
    t2 = (beta2 - mean2 * (gamma2 / jnp.sqrt(var2 + _EPS))).reshape(1, F)
    b1r = b1.reshape(1, F)
    b2r = b2.reshape(1, F)
    args = (x, W1, b1r, s1, t1, W2, b2r, s2, t2)

    devs = jax.devices()
    nd = max(d for d in (1, 2, 4) if d <= len(devs) and B % d == 0)
    if nd == 1:
        return _edgeconv_shard(*args)
    mesh = jax.sharding.Mesh(np.array(devs[:nd]), ("d",))
    P = jax.sharding.PartitionSpec
    rep = P()
    fn = jax.shard_map(
        _edgeconv_shard,
        mesh=mesh,
        in_specs=(P("d"), rep, rep, rep, rep, rep, rep, rep, rep),
        out_specs=P("d"),
        check_vma=False,
    )
    return fn(*args)
